# baseline (speedup 1.0000x reference)
import functools

import jax
import jax.numpy as jnp
from jax import lax
from jax.experimental import pallas as pl
from jax.experimental.pallas import tpu as pltpu
from jax.experimental.pallas import tpu_sc as plsc

TOKEN = 32
HID = 768
NSC = 1
NW = NSC * 16
LANES = 16
SBLK = 128


def _tc_emb_kernel(bits_ref, table_ref, emb_ref):
    amp = bits_ref[...] * 2.0 - 1.0
    emb_ref[...] = amp[:, :, None] * table_ref[...][None, :, :]


def _sc_logit_kernel(bits_hbm, tableT_hbm, logit_hbm, tab_v, bits_v, logit_v):
    wid = lax.axis_index("s") * NSC + lax.axis_index("c")
    n_vals = bits_hbm.shape[0]
    vals_pw = n_vals // NW
    v0 = wid * vals_pw

    pltpu.sync_copy(tableT_hbm, tab_v)
    pltpu.sync_copy(bits_hbm.at[pl.ds(v0, vals_pw)], bits_v)

    RS_UNROLL = 8

    def rsum_body(g, accs):
        acc_lo, acc_hi = accs
        for u in range(RS_UNROLL):
            o = (g * RS_UNROLL + u) * TOKEN
            acc_lo = acc_lo + tab_v[pl.ds(o, LANES)]
            acc_hi = acc_hi + tab_v[pl.ds(o + LANES, LANES)]
        return acc_lo, acc_hi

    zeros = jnp.zeros((LANES,), jnp.float32)
    rs_lo, rs_hi = lax.fori_loop(0, HID // RS_UNROLL, rsum_body, (zeros, zeros))

    def logit_body(k, _):
        o = k * TOKEN
        b_lo = bits_v[pl.ds(o, LANES)]
        logit_v[pl.ds(o, LANES)] = (b_lo * 2.0 - 1.0) * rs_lo
        b_hi = bits_v[pl.ds(o + LANES, LANES)]
        logit_v[pl.ds(o + LANES, LANES)] = (b_hi * 2.0 - 1.0) * rs_hi
        return 0

    lax.fori_loop(0, vals_pw // TOKEN, logit_body, 0)
    pltpu.sync_copy(logit_v, logit_hbm.at[pl.ds(v0, vals_pw)])


def kernel(text_batch, table):
    B, flat = text_batch.shape
    S = flat // TOKEN
    N = B * S
    bits = text_batch.reshape(N, TOKEN)

    emb = pl.pallas_call(
        _tc_emb_kernel,
        grid=(N // SBLK,),
        in_specs=[
            pl.BlockSpec((SBLK, TOKEN), lambda i: (i, 0)),
            pl.BlockSpec((TOKEN, HID), lambda i: (0, 0)),
        ],
        out_specs=pl.BlockSpec((SBLK, TOKEN, HID), lambda i: (i, 0, 0)),
        out_shape=jax.ShapeDtypeStruct((N, TOKEN, HID), jnp.float32),
    )(bits, table)

    mesh = plsc.VectorSubcoreMesh(core_axis_name="c", subcore_axis_name="s",
                                  num_cores=NSC)
    sc_call = functools.partial(
        pl.kernel,
        mesh=mesh,
        out_type=jax.ShapeDtypeStruct((N * TOKEN,), jnp.float32),
        scratch_types=[
            pltpu.VMEM((TOKEN * HID,), jnp.float32),
            pltpu.VMEM((N * TOKEN // NW,), jnp.float32),
            pltpu.VMEM((N * TOKEN // NW,), jnp.float32),
        ],
        cost_estimate=pl.CostEstimate(
            flops=2 * N * TOKEN, bytes_accessed=8 * N * TOKEN,
            transcendentals=0),
    )(_sc_logit_kernel)
    logit = sc_call(text_batch.reshape(N * TOKEN), table.T.reshape(TOKEN * HID))

    return emb.reshape(B, S, TOKEN, HID), logit.reshape(B, S, TOKEN, 1)


# speedup vs baseline: 1.0171x; 1.0171x over previous
import functools

import jax
import jax.numpy as jnp
from jax import lax
from jax.experimental import pallas as pl
from jax.experimental.pallas import tpu as pltpu
from jax.experimental.pallas import tpu_sc as plsc

TOKEN = 32
HID = 768
NSC = 1
NSUB = 8
NW = NSC * NSUB
LANES = 16
SBLK = 64


def _tc_emb_kernel(bits_ref, table_ref, emb_ref):
    amp = bits_ref[...] * 2.0 - 1.0
    emb_ref[...] = amp[:, :, None] * table_ref[...][None, :, :]


def _sc_logit_kernel(bits_hbm, tableT_hbm, logit_hbm, tab_v, bits_v, logit_v):
    wid = lax.axis_index("s") * NSC + lax.axis_index("c")
    n_vals = bits_hbm.shape[0]
    vals_pw = n_vals // NW
    v0 = wid * vals_pw

    pltpu.sync_copy(tableT_hbm, tab_v)
    pltpu.sync_copy(bits_hbm.at[pl.ds(v0, vals_pw)], bits_v)

    RS_UNROLL = 8

    def rsum_body(g, accs):
        acc_lo, acc_hi = accs
        for u in range(RS_UNROLL):
            o = (g * RS_UNROLL + u) * TOKEN
            acc_lo = acc_lo + tab_v[pl.ds(o, LANES)]
            acc_hi = acc_hi + tab_v[pl.ds(o + LANES, LANES)]
        return acc_lo, acc_hi

    zeros = jnp.zeros((LANES,), jnp.float32)
    rs_lo, rs_hi = lax.fori_loop(0, HID // RS_UNROLL, rsum_body, (zeros, zeros))

    def logit_body(k, _):
        o = k * TOKEN
        b_lo = bits_v[pl.ds(o, LANES)]
        logit_v[pl.ds(o, LANES)] = (b_lo * 2.0 - 1.0) * rs_lo
        b_hi = bits_v[pl.ds(o + LANES, LANES)]
        logit_v[pl.ds(o + LANES, LANES)] = (b_hi * 2.0 - 1.0) * rs_hi
        return 0

    lax.fori_loop(0, vals_pw // TOKEN, logit_body, 0)
    pltpu.sync_copy(logit_v, logit_hbm.at[pl.ds(v0, vals_pw)])


def kernel(text_batch, table):
    B, flat = text_batch.shape
    S = flat // TOKEN
    N = B * S
    bits = text_batch.reshape(N, TOKEN)

    emb = pl.pallas_call(
        _tc_emb_kernel,
        grid=(N // SBLK,),
        in_specs=[
            pl.BlockSpec((SBLK, TOKEN), lambda i: (i, 0)),
            pl.BlockSpec((TOKEN, HID), lambda i: (0, 0)),
        ],
        out_specs=pl.BlockSpec((SBLK, TOKEN, HID), lambda i: (i, 0, 0)),
        out_shape=jax.ShapeDtypeStruct((N, TOKEN, HID), jnp.float32),
    )(bits, table)

    mesh = plsc.VectorSubcoreMesh(core_axis_name="c", subcore_axis_name="s",
                                  num_cores=NSC, num_subcores=NSUB)
    sc_call = functools.partial(
        pl.kernel,
        mesh=mesh,
        out_type=jax.ShapeDtypeStruct((N * TOKEN,), jnp.float32),
        scratch_types=[
            pltpu.VMEM((TOKEN * HID,), jnp.float32),
            pltpu.VMEM((N * TOKEN // NW,), jnp.float32),
            pltpu.VMEM((N * TOKEN // NW,), jnp.float32),
        ],
        cost_estimate=pl.CostEstimate(
            flops=2 * N * TOKEN, bytes_accessed=8 * N * TOKEN,
            transcendentals=0),
    )(_sc_logit_kernel)
    logit = sc_call(text_batch.reshape(N * TOKEN), table.T.reshape(TOKEN * HID))

    return emb.reshape(B, S, TOKEN, HID), logit.reshape(B, S, TOKEN, 1)
